# Initial kernel scaffold; baseline (speedup 1.0000x reference)
import jax
import jax.numpy as jnp
from jax.experimental import pallas as pl


def kernel(input, rpq_indices, codebooks):
    raise NotImplementedError("write your pallas kernel here")


# speedup vs baseline: 719.6320x; 719.6320x over previous
import functools

import jax
import jax.numpy as jnp
from jax import lax
from jax.experimental import pallas as pl
from jax.experimental.pallas import tpu as pltpu
from jax.experimental.pallas import tpu_sc as plsc

NEMB = 1_000_000
NCB = 8
CBD = 8
NCODES = 256
D = NCB * CBD
N = 4096 * 50

NW = 32
N_W = N // NW
CHUNK = 640
NCHUNK = N_W // CHUNK
IDXB = 128
NB = CHUNK * NCB // IDXB
GGRP = CHUNK // 16

_mesh = plsc.VectorSubcoreMesh(core_axis_name="c", subcore_axis_name="s")


@functools.partial(
    pl.kernel,
    mesh=_mesh,
    compiler_params=pltpu.CompilerParams(needs_layout_passes=False),
    out_type=jax.ShapeDtypeStruct((N * D,), jnp.float32),
    scratch_types=[
        pltpu.VMEM((N_W,), jnp.int32),
        pltpu.VMEM((CHUNK * NCB,), jnp.int32),
        pltpu.VMEM((CHUNK * NCB,), jnp.int32),
        pltpu.VMEM((NCB * NCODES * CBD,), jnp.float32),
        pltpu.VMEM((CHUNK * D,), jnp.float32),
        pltpu.SemaphoreType.DMA,
    ],
)
def _rpq_sc(ids_hbm, rpq_hbm, cb_hbm, out_hbm, ids_v, ibuf, codes_v, cb_v,
            out_v, gsem):
    wid = lax.axis_index("c") * 16 + lax.axis_index("s")
    base = wid * N_W

    pltpu.sync_copy(ids_hbm.at[pl.ds(base, N_W)], ids_v)
    pltpu.sync_copy(cb_hbm, cb_v)

    lane = lax.iota(jnp.int32, 16)
    lane_h = lane & 7
    lane_pair = lane >> 3
    hterm = lane_h * NEMB
    hoff = lane_h * (NCODES * CBD)
    opat = lane_pair * D + lane_h * CBD

    def chunk_body(c, carry):
        def build(v, inner):
            idv = plsc.load_gather(ids_v, [c * CHUNK + v * 2 + lane_pair])
            ibuf[pl.ds(v * 16, 16)] = idv + hterm
            return inner

        lax.fori_loop(0, CHUNK * NCB // 16, build, 0)

        copies = [
            pltpu.async_copy(
                rpq_hbm.at[ibuf.at[pl.ds(b * IDXB, IDXB)]],
                codes_v.at[pl.ds(b * IDXB, IDXB)],
                gsem,
            )
            for b in range(NB)
        ]
        for cp in copies:
            cp.wait()

        def group_body(g, inner):
            for v8 in range(8):
                cvec = codes_v[pl.ds(g * 128 + v8 * 16, 16)]
                cb_idx = hoff + cvec * CBD
                obase = (g * 16 + v8 * 2) * D
                for d in range(CBD):
                    val = plsc.load_gather(cb_v, [cb_idx + d])
                    plsc.store_scatter(out_v, [obase + opat + d], val)
            return inner

        lax.fori_loop(0, GGRP, group_body, 0)

        pltpu.sync_copy(
            out_v, out_hbm.at[pl.ds((base + c * CHUNK) * D, CHUNK * D)])
        return carry

    lax.fori_loop(0, NCHUNK, chunk_body, 0)


def kernel(input, rpq_indices, codebooks):
    ids = input.reshape(-1)
    rpqf = rpq_indices.reshape(-1)
    cbf = codebooks.reshape(-1)
    out = _rpq_sc(ids, rpqf, cbf)
    return out.reshape(input.shape + (D,))


# speedup vs baseline: 1273.7914x; 1.7701x over previous
import functools

import jax
import jax.numpy as jnp
from jax import lax
from jax.experimental import pallas as pl
from jax.experimental.pallas import tpu as pltpu
from jax.experimental.pallas import tpu_sc as plsc

NCB = 8
CBD = 8
NCODES = 256
D = NCB * CBD
N = 4096 * 50

NW = 32
N_W = N // NW
CH = 800
NCHUNK = N_W // CH
BATCHES = [(k * 128, 128) for k in range(6)] + [(768, 32)]
GGRP = CH // 16

_mesh = plsc.VectorSubcoreMesh(core_axis_name="c", subcore_axis_name="s")


@functools.partial(
    pl.kernel,
    mesh=_mesh,
    compiler_params=pltpu.CompilerParams(needs_layout_passes=False),
    out_type=jax.ShapeDtypeStruct((N * D,), jnp.float32),
    scratch_types=[
        pltpu.VMEM((N_W,), jnp.int32),
        pltpu.VMEM((CH,), jnp.int32),
        pltpu.VMEM((CH,), jnp.int32),
        pltpu.VMEM((NCB * NCODES * CBD,), jnp.float32),
        pltpu.VMEM((CH * D,), jnp.float32),
        pltpu.SemaphoreType.DMA,
    ],
)
def _rpq_sc(ids_hbm, w0_hbm, w1_hbm, cb_hbm, out_hbm, ids_v, codes0, codes1,
            cb_v, out_v, gsem):
    wid = lax.axis_index("c") * 16 + lax.axis_index("s")
    base = wid * N_W

    pltpu.sync_copy(ids_hbm.at[pl.ds(base, N_W)], ids_v)
    pltpu.sync_copy(cb_hbm, cb_v)

    lane = lax.iota(jnp.int32, 16)

    def chunk_body(c, carry):
        copies = [
            pltpu.async_copy(
                tbl.at[ids_v.at[pl.ds(c * CH + off, sz)]],
                dst.at[pl.ds(off, sz)],
                gsem,
            )
            for tbl, dst in ((w0_hbm, codes0), (w1_hbm, codes1))
            for off, sz in BATCHES
        ]
        for cp in copies:
            cp.wait()

        def group_body(v, inner):
            cw = (codes0[pl.ds(v * 16, 16)], codes1[pl.ds(v * 16, 16)])
            obase = (v * 16 + lane) * D
            for h in range(NCB):
                code = (cw[h // 4] >> (8 * (h % 4))) & 255
                cb_idx = code * CBD + h * (NCODES * CBD)
                for d in range(CBD):
                    val = plsc.load_gather(cb_v, [cb_idx + d])
                    plsc.store_scatter(out_v, [obase + h * CBD + d], val)
            return inner

        lax.fori_loop(0, GGRP, group_body, 0)

        pltpu.sync_copy(
            out_v, out_hbm.at[pl.ds((base + c * CH) * D, CH * D)])
        return carry

    lax.fori_loop(0, NCHUNK, chunk_body, 0)


def kernel(input, rpq_indices, codebooks):
    ids = input.reshape(-1)
    r = rpq_indices
    w0 = r[0] | (r[1] << 8) | (r[2] << 16) | (r[3] << 24)
    w1 = r[4] | (r[5] << 8) | (r[6] << 16) | (r[7] << 24)
    cbf = codebooks.reshape(-1)
    out = _rpq_sc(ids, w0, w1, cbf)
    return out.reshape(input.shape + (D,))


# speedup vs baseline: 1332.9804x; 1.0465x over previous
import functools

import jax
import jax.numpy as jnp
from jax import lax
from jax.experimental import pallas as pl
from jax.experimental.pallas import tpu as pltpu
from jax.experimental.pallas import tpu_sc as plsc

NCB = 8
CBD = 8
NCODES = 256
D = NCB * CBD
N = 4096 * 50

NW = 32
N_W = N // NW
CH = 800
NCHUNK = N_W // CH
BATCHES = [(k * 128, 128) for k in range(6)] + [(768, 32)]
GGRP = CH // 16

_mesh = plsc.VectorSubcoreMesh(core_axis_name="c", subcore_axis_name="s")


@functools.partial(
    pl.kernel,
    mesh=_mesh,
    compiler_params=pltpu.CompilerParams(needs_layout_passes=False),
    out_type=jax.ShapeDtypeStruct((N * D,), jnp.float32),
    scratch_types=[
        pltpu.VMEM((N_W,), jnp.int32),
        pltpu.VMEM((CH,), jnp.int32),
        pltpu.VMEM((CH,), jnp.int32),
        pltpu.VMEM((CH,), jnp.int32),
        pltpu.VMEM((CH,), jnp.int32),
        pltpu.VMEM((NCB * NCODES * CBD,), jnp.float32),
        pltpu.VMEM((CH * D,), jnp.float32),
        pltpu.VMEM((CH * D,), jnp.float32),
        pltpu.SemaphoreType.DMA,
        pltpu.SemaphoreType.DMA,
        pltpu.SemaphoreType.DMA,
        pltpu.SemaphoreType.DMA,
    ],
)
def _rpq_sc(ids_hbm, w0_hbm, w1_hbm, cb_hbm, out_hbm, ids_v, c0a, c0b,
            c1a, c1b, cb_v, outa, outb, gsem0, gsem1, osem0, osem1):
    wid = lax.axis_index("c") * 16 + lax.axis_index("s")
    base = wid * N_W
    codes0 = (c0a, c0b)
    codes1 = (c1a, c1b)
    out_v = (outa, outb)
    gsems = (gsem0, gsem1)
    osems = (osem0, osem1)

    pltpu.sync_copy(ids_hbm.at[pl.ds(base, N_W)], ids_v)
    pltpu.sync_copy(cb_hbm, cb_v)

    lane = lax.iota(jnp.int32, 16)

    def fire_gathers(c):
        p = c % 2
        return [
            pltpu.async_copy(
                tbl.at[ids_v.at[pl.ds(c * CH + off, sz)]],
                dst[p].at[pl.ds(off, sz)],
                gsems[p],
            )
            for tbl, dst in ((w0_hbm, codes0), (w1_hbm, codes1))
            for off, sz in BATCHES
        ]

    out_copies = {}
    pending = fire_gathers(0)
    for c in range(NCHUNK):
        p = c % 2
        nxt = fire_gathers(c + 1) if c + 1 < NCHUNK else []
        for cp in pending:
            cp.wait()
        pending = nxt

        if c >= 2:
            out_copies[c - 2].wait()

        def group_body(v, inner):
            cw = (codes0[p][pl.ds(v * 16, 16)], codes1[p][pl.ds(v * 16, 16)])
            obase = (v * 16 + lane) * D
            for h in range(NCB):
                code = (cw[h // 4] >> (8 * (h % 4))) & 255
                cb_idx = code * CBD + h * (NCODES * CBD)
                for d in range(CBD):
                    val = plsc.load_gather(cb_v, [cb_idx + d])
                    plsc.store_scatter(out_v[p], [obase + h * CBD + d], val)
            return inner

        lax.fori_loop(0, GGRP, group_body, 0)

        out_copies[c] = pltpu.async_copy(
            out_v[p], out_hbm.at[pl.ds((base + c * CH) * D, CH * D)],
            osems[p])

    out_copies[NCHUNK - 2].wait()
    out_copies[NCHUNK - 1].wait()


def kernel(input, rpq_indices, codebooks):
    ids = input.reshape(-1)
    r = rpq_indices
    w0 = r[0] | (r[1] << 8) | (r[2] << 16) | (r[3] << 24)
    w1 = r[4] | (r[5] << 8) | (r[6] << 16) | (r[7] << 24)
    cbf = codebooks.reshape(-1)
    out = _rpq_sc(ids, w0, w1, cbf)
    return out.reshape(input.shape + (D,))


# speedup vs baseline: 2192.1518x; 1.6445x over previous
import functools

import jax
import jax.numpy as jnp
from jax import lax
from jax.experimental import pallas as pl
from jax.experimental.pallas import tpu as pltpu
from jax.experimental.pallas import tpu_sc as plsc

NCB = 8
CBD = 8
NCODES = 256
D = NCB * CBD
N = 4096 * 50

NW = 32
N_W = N // NW
CH = 800
NCHUNK = N_W // CH
BATCHES = [(k * 128, 128) for k in range(6)] + [(768, 32)]
GGRP = CH // 16

_mesh = plsc.VectorSubcoreMesh(core_axis_name="c", subcore_axis_name="s")


@functools.partial(
    pl.kernel,
    mesh=_mesh,
    compiler_params=pltpu.CompilerParams(needs_layout_passes=False),
    out_type=jax.ShapeDtypeStruct((N * D,), jnp.float32),
    scratch_types=[
        pltpu.VMEM((N_W,), jnp.int32),
        pltpu.VMEM((CH,), jnp.int32),
        pltpu.VMEM((CH,), jnp.int32),
        pltpu.VMEM((CH,), jnp.int32),
        pltpu.VMEM((CH,), jnp.int32),
        pltpu.VMEM((NCB * NCODES * CBD,), jnp.float32),
        pltpu.VMEM((CH * D,), jnp.float32),
        pltpu.VMEM((CH * D,), jnp.float32),
        pltpu.SemaphoreType.DMA,
        pltpu.SemaphoreType.DMA,
        pltpu.SemaphoreType.DMA,
        pltpu.SemaphoreType.DMA,
    ],
)
def _rpq_sc(ids_hbm, w0_hbm, w1_hbm, cb_hbm, out_hbm, ids_v, c0a, c0b,
            c1a, c1b, cb_v, outa, outb, gsem0, gsem1, osem0, osem1):
    wid = lax.axis_index("c") * 16 + lax.axis_index("s")
    base = wid * N_W
    codes0 = (c0a, c0b)
    codes1 = (c1a, c1b)
    out_v = (outa, outb)
    gsems = (gsem0, gsem1)
    osems = (osem0, osem1)

    pltpu.sync_copy(ids_hbm.at[pl.ds(base, N_W)], ids_v)
    pltpu.sync_copy(cb_hbm, cb_v)

    lane = lax.iota(jnp.int32, 16)
    half = lane >> 3
    shift_even = half * 8
    shift_odd = 16 + half * 8
    cbase = [(2 * k + half) * (NCODES * CBD) + (lane & 7) for k in range(4)]

    def fire_gathers(c):
        p = c % 2
        return [
            pltpu.async_copy(
                tbl.at[ids_v.at[pl.ds(c * CH + off, sz)]],
                dst[p].at[pl.ds(off, sz)],
                gsems[p],
            )
            for tbl, dst in ((w0_hbm, codes0), (w1_hbm, codes1))
            for off, sz in BATCHES
        ]

    out_copies = {}
    pending = fire_gathers(0)
    for c in range(NCHUNK):
        p = c % 2
        nxt = fire_gathers(c + 1) if c + 1 < NCHUNK else []
        for cp in pending:
            cp.wait()
        pending = nxt

        if c >= 2:
            out_copies[c - 2].wait()

        def group_body(v, inner):
            cw0 = codes0[p][pl.ds(v * 16, 16)]
            cw1 = codes1[p][pl.ds(v * 16, 16)]
            for j in range(16):
                w0s = jnp.broadcast_to(cw0[j], (16,))
                w1s = jnp.broadcast_to(cw1[j], (16,))
                ob = (v * 16 + j) * D
                for k in range(4):
                    w = w0s if k < 2 else w1s
                    shift = shift_even if k % 2 == 0 else shift_odd
                    code = (w >> shift) & 255
                    val = plsc.load_gather(cb_v, [(code << 3) + cbase[k]])
                    out_v[p][pl.ds(ob + k * 16, 16)] = val
            return inner

        lax.fori_loop(0, GGRP, group_body, 0)

        out_copies[c] = pltpu.async_copy(
            out_v[p], out_hbm.at[pl.ds((base + c * CH) * D, CH * D)],
            osems[p])

    out_copies[NCHUNK - 2].wait()
    out_copies[NCHUNK - 1].wait()


def kernel(input, rpq_indices, codebooks):
    ids = input.reshape(-1)
    r = rpq_indices
    w0 = r[0] | (r[1] << 8) | (r[2] << 16) | (r[3] << 24)
    w1 = r[4] | (r[5] << 8) | (r[6] << 16) | (r[7] << 24)
    cbf = codebooks.reshape(-1)
    out = _rpq_sc(ids, w0, w1, cbf)
    return out.reshape(input.shape + (D,))
